# baseline (speedup 1.0000x reference)
import functools

import jax
import jax.numpy as jnp
from jax import lax
from jax.experimental import pallas as pl
from jax.experimental.pallas import tpu as pltpu
from jax.experimental.pallas import tpu_sc as plsc

T, N, F0, F1, F2 = 4, 10000, 128, 64, 32
NNZ, E = 160000, 200000
NC, NS, L = 2, 16, 16
NW = NC * NS
TN = T * N

PER_TILE = NNZ // NS
CH = 80
NCHUNK = PER_TILE // CH
RPT = N // NS

CH2 = 80
NCHUNK2 = E // CH2


def _mm_y_body(x_ref, w_ref, u_ref, o_ref):
    w = w_ref[...]
    g = jnp.concatenate(
        [jnp.dot(w, u_ref[:F1, :], preferred_element_type=jnp.float32),
         jnp.dot(w, u_ref[F1:, :], preferred_element_type=jnp.float32)],
        axis=1)
    o_ref[...] = jnp.dot(x_ref[...], g, preferred_element_type=jnp.float32)


def _tc_y(Xf, W1, U):
    BM = 2000
    return pl.pallas_call(
        _mm_y_body,
        grid=(TN // BM,),
        in_specs=[
            pl.BlockSpec((BM, F0), lambda i: (i, 0)),
            pl.BlockSpec((F0, F1), lambda i: (0, 0)),
            pl.BlockSpec((2 * F1, F2), lambda i: (0, 0)),
        ],
        out_specs=pl.BlockSpec((BM, 2 * F2), lambda i: (i, 0)),
        out_shape=jax.ShapeDtypeStruct((TN, 2 * F2), jnp.float32),
    )(Xf, W1, U)


ZROWS = 200
CROWS = 2000


def _seg_body(aidx, aval, xw, z_out,
              craw_b, didx_b, val_b,
              gidx0, gidx1, gidx2, gidx3,
              didxc0, didxc1, didxc2, didxc3,
              rows0, rows1, rows2, rows3,
              scaled0, scaled1, scaled2, scaled3, zb_v,
              zsh,
              bsem, gsem0, gsem1, gsem2, gsem3,
              ssem0, ssem1, ssem2, ssem3):
    c = lax.axis_index("c")
    s = lax.axis_index("s")
    iota = lax.iota(jnp.int32, L)
    zero = jnp.zeros((L,), jnp.float32)
    gidx = (gidx0, gidx1, gidx2, gidx3)
    didxc = (didxc0, didxc1, didxc2, didxc3)
    rows = (rows0, rows1, rows2, rows3)
    scaled = (scaled0, scaled1, scaled2, scaled3)
    gsem = (gsem0, gsem1, gsem2, gsem3)
    ssem = (ssem0, ssem1, ssem2, ssem3)

    def zfill(i, carry):
        for j in range(F1 // L):
            zb_v[i, pl.ds(j * L, L)] = zero
        return carry

    lax.fori_loop(0, ZROWS, zfill, 0)

    def zero_my_rows():
        @pl.when(s < N // CROWS)
        def _():
            for k in range(CROWS // ZROWS):
                rws = pl.ds(s * CROWS + k * ZROWS, ZROWS)
                pltpu.sync_copy(zb_v, zsh.at[rws])

    zero_my_rows()
    plsc.subcore_barrier()

    for tt in (0, 1):
        t = c * 2 + tt
        tn_vec = jnp.full((L,), t * N, jnp.int32)

        nzs = pl.ds(s * PER_TILE, PER_TILE)
        d1 = pltpu.async_copy(aidx.at[t, 1, nzs], craw_b, bsem)
        d2 = pltpu.async_copy(aidx.at[t, 0, nzs], didx_b, bsem)
        d3 = pltpu.async_copy(aval.at[t, nzs], val_b, bsem)
        d1.wait()
        d2.wait()
        d3.wait()

        def prep_and_gather(j, slot):
            off = j * CH
            for g in range(CH // L):
                d = pl.ds(g * L, L)
                src = pl.ds(off + g * L, L)
                gidx[slot][d] = craw_b[src] + tn_vec
                didxc[slot][d] = didx_b[src]
            pltpu.async_copy(xw.at[gidx[slot]], rows[slot], gsem[slot])

        def wait_gather(slot):
            pltpu.make_async_copy(xw.at[gidx[slot]], rows[slot],
                                  gsem[slot]).wait()

        def scale(j, slot):
            off = j * CH
            rows_s = rows[slot]
            scaled_s = scaled[slot]

            for g in range(CH // L):
                vv = val_b[pl.ds(off + g * L, L)]

                @plsc.parallel_loop(g * L, (g + 1) * L, unroll=8)
                def _(r):
                    sv = lax.gather(
                        vv, jnp.full((L, 1), r - g * L, jnp.int32),
                        lax.GatherDimensionNumbers(
                            offset_dims=(), collapsed_slice_dims=(0,),
                            start_index_map=(0,)),
                        (1,), mode=lax.GatherScatterMode.PROMISE_IN_BOUNDS)
                    for jb in range(F1 // L):
                        d = pl.ds(jb * L, L)
                        scaled_s[r, d] = rows_s[r, d] * sv

        def scatter_add(slot):
            pltpu.async_copy(scaled[slot], zsh.at[didxc[slot]], ssem[slot],
                             add=True)

        def wait_scatter(slot):
            pltpu.make_async_copy(scaled[slot], zsh.at[didxc[slot]],
                                  ssem[slot]).wait()

        prep_and_gather(0, 0)
        prep_and_gather(1, 1)
        prep_and_gather(2, 2)

        def pipe(m, carry):
            for u in range(4):
                j = 4 * m + u

                @pl.when(j >= 1)
                def _():
                    wait_scatter((u + 3) % 4)

                @pl.when(j + 3 <= NCHUNK - 1)
                def _():
                    prep_and_gather(j + 3, (u + 3) % 4)

                wait_gather(u)
                scale(j, u)
                scatter_add(u)
            return carry

        lax.fori_loop(0, (NCHUNK - 1) // 4, pipe, 0)
        wait_scatter(3)
        wait_gather(0)
        scale(NCHUNK - 1, 0)
        scatter_add(0)
        wait_scatter(0)

        plsc.subcore_barrier()

        @pl.when(s < N // CROWS)
        def _():
            rws = pl.ds(s * CROWS, CROWS)
            pltpu.sync_copy(zsh.at[rws],
                            z_out.at[pl.ds(t * N + s * CROWS, CROWS)])

        if tt == 0:
            zero_my_rows()
            plsc.subcore_barrier()


def _sc_seg(aidx, aval, XW):
    mesh = plsc.VectorSubcoreMesh(core_axis_name="c", subcore_axis_name="s",
                                  num_cores=NC, num_subcores=NS)
    f = pl.kernel(
        _seg_body,
        out_type=jax.ShapeDtypeStruct((TN, F1), jnp.float32),
        mesh=mesh,
        compiler_params=pltpu.CompilerParams(needs_layout_passes=False, use_tc_tiling_on_sc=False),
        scratch_types=[
            pltpu.VMEM((PER_TILE,), jnp.int32),
            pltpu.VMEM((PER_TILE,), jnp.int32),
            pltpu.VMEM((PER_TILE,), jnp.float32),
            pltpu.VMEM((CH,), jnp.int32),
            pltpu.VMEM((CH,), jnp.int32),
            pltpu.VMEM((CH,), jnp.int32),
            pltpu.VMEM((CH,), jnp.int32),
            pltpu.VMEM((CH,), jnp.int32),
            pltpu.VMEM((CH,), jnp.int32),
            pltpu.VMEM((CH,), jnp.int32),
            pltpu.VMEM((CH,), jnp.int32),
            pltpu.VMEM((CH, F1), jnp.float32),
            pltpu.VMEM((CH, F1), jnp.float32),
            pltpu.VMEM((CH, F1), jnp.float32),
            pltpu.VMEM((CH, F1), jnp.float32),
            pltpu.VMEM((CH, F1), jnp.float32),
            pltpu.VMEM((CH, F1), jnp.float32),
            pltpu.VMEM((CH, F1), jnp.float32),
            pltpu.VMEM((CH, F1), jnp.float32),
            pltpu.VMEM((ZROWS, F1), jnp.float32),
            pltpu.VMEM_SHARED((N, F1), jnp.float32),
            pltpu.SemaphoreType.DMA,
            pltpu.SemaphoreType.DMA,
            pltpu.SemaphoreType.DMA,
            pltpu.SemaphoreType.DMA,
            pltpu.SemaphoreType.DMA,
            pltpu.SemaphoreType.DMA,
            pltpu.SemaphoreType.DMA,
            pltpu.SemaphoreType.DMA,
            pltpu.SemaphoreType.DMA,
        ],
    )
    return f(aidx, aval, XW)


def _edge_body(pq, et, es, eg, out,
               tb0, tb1, tb2, sb0, sb1, sb2, gb0, gb1, gb2,
               sidx0, sidx1, sidx2, tidx0, tidx1, tidx2,
               pb0, pb1, pb2, qb0, qb1, qb2, ob0, ob1, ob2,
               isem0, isem1, isem2, psem0, psem1, psem2,
               qsem0, qsem1, qsem2, osem0, osem1, osem2):
    c = lax.axis_index("c")
    s = lax.axis_index("s")
    wid = s * NC + c
    nch = (NCHUNK2 - wid + NW - 1) // NW
    tb = (tb0, tb1, tb2)
    sb = (sb0, sb1, sb2)
    gb = (gb0, gb1, gb2)
    sidx = (sidx0, sidx1, sidx2)
    tidx = (tidx0, tidx1, tidx2)
    pb = (pb0, pb1, pb2)
    qb = (qb0, qb1, qb2)
    ob = (ob0, ob1, ob2)
    isem = (isem0, isem1, isem2)
    psem = (psem0, psem1, psem2)
    qsem = (qsem0, qsem1, qsem2)
    osem = (osem0, osem1, osem2)

    def off_of(k):
        return (wid + k * NW) * CH2

    def idx_load(k, u):
        o = pl.ds(off_of(k), CH2)
        pltpu.async_copy(et.at[o], tb[u], isem[u])
        pltpu.async_copy(es.at[o], sb[u], isem[u])
        pltpu.async_copy(eg.at[o], gb[u], isem[u])

    def idx_wait(u):
        o = pl.ds(0, CH2)
        pltpu.make_async_copy(et.at[o], tb[u], isem[u]).wait()
        pltpu.make_async_copy(es.at[o], sb[u], isem[u]).wait()
        pltpu.make_async_copy(eg.at[o], gb[u], isem[u]).wait()

    def ids_and_gather(u):
        for g in range(CH2 // L):
            d = pl.ds(g * L, L)
            tv = tb[u][d] * N
            sidx[u][d] = tv + sb[u][d]
            tidx[u][d] = tv + gb[u][d]
        pltpu.async_copy(pq.at[sidx[u]], pb[u], psem[u])
        pltpu.async_copy(pq.at[tidx[u]], qb[u], qsem[u])

    def gather_wait(u):
        pltpu.make_async_copy(pq.at[sidx[u]], pb[u], psem[u]).wait()
        pltpu.make_async_copy(pq.at[tidx[u]], qb[u], qsem[u]).wait()

    def out_wait(u):
        pltpu.make_async_copy(ob[u], out.at[pl.ds(0, CH2)], osem[u]).wait()

    idx_load(0, 0)
    idx_load(1, 1)
    idx_wait(0)
    ids_and_gather(0)

    def pipe(m, carry):
        for u in range(3):
            k = 3 * m + u

            @pl.when(k < nch)
            def _():
                gather_wait(u)

                @pl.when(k + 1 < nch)
                def _():
                    idx_wait((u + 1) % 3)
                    ids_and_gather((u + 1) % 3)

                @pl.when(k + 2 < nch)
                def _():
                    idx_load(k + 2, (u + 2) % 3)

                @pl.when(k >= 3)
                def _():
                    out_wait(u)

                for r in range(CH2):
                    for jj in range(F2 // L):
                        dd = pl.ds(jj * L, L)
                        dq = pl.ds(F2 + jj * L, L)
                        ob[u][r, dd] = pb[u][r, dd] + qb[u][r, dq]
                pltpu.async_copy(ob[u], out.at[pl.ds(off_of(k), CH2)],
                                 osem[u])

        return carry

    lax.fori_loop(0, (nch + 2) // 3, pipe, 0)
    out_wait(0)
    out_wait(1)
    out_wait(2)


def _sc_edge(PQ, et, es, eg):
    mesh = plsc.VectorSubcoreMesh(core_axis_name="c", subcore_axis_name="s",
                                  num_cores=NC, num_subcores=NS)
    f = pl.kernel(
        _edge_body,
        out_type=jax.ShapeDtypeStruct((E, F2), jnp.float32),
        mesh=mesh,
        compiler_params=pltpu.CompilerParams(needs_layout_passes=False, use_tc_tiling_on_sc=False),
        scratch_types=(
            [pltpu.VMEM((CH2,), jnp.int32)] * 15
            + [pltpu.VMEM((CH2, 2 * F2), jnp.float32)] * 6
            + [pltpu.VMEM((CH2, F2), jnp.float32)] * 3
            + [pltpu.SemaphoreType.DMA] * 12
        ),
    )
    return f(PQ, et, es, eg)


def kernel(A_idx, A_val, X, edges_t, edges_src, edges_trg, W1, U):
    aidx = A_idx.astype(jnp.int32)
    Xf = X.reshape(TN, F0)
    Y = _tc_y(Xf, W1, U)
    PQ = _sc_seg(aidx, A_val, Y)
    return _sc_edge(PQ,
                    edges_t.astype(jnp.int32),
                    edges_src.astype(jnp.int32),
                    edges_trg.astype(jnp.int32))


# speedup vs baseline: 1.0380x; 1.0380x over previous
import functools

import jax
import jax.numpy as jnp
from jax import lax
from jax.experimental import pallas as pl
from jax.experimental.pallas import tpu as pltpu
from jax.experimental.pallas import tpu_sc as plsc

T, N, F0, F1, F2 = 4, 10000, 128, 64, 32
NNZ, E = 160000, 200000
NC, NS, L = 2, 16, 16
NW = NC * NS
TN = T * N

PER_TILE = NNZ // NS
CH = 80
NCHUNK = PER_TILE // CH
RPT = N // NS

CH2 = 80
NCHUNK2 = E // CH2


def _mm_y_body(x_ref, w_ref, u_ref, o_ref):
    w = w_ref[...]
    g = jnp.concatenate(
        [jnp.dot(w, u_ref[:F1, :], preferred_element_type=jnp.float32),
         jnp.dot(w, u_ref[F1:, :], preferred_element_type=jnp.float32)],
        axis=1)
    o_ref[...] = jnp.dot(x_ref[...], g, preferred_element_type=jnp.float32)


def _tc_y(Xf, W1, U):
    BM = 2000
    return pl.pallas_call(
        _mm_y_body,
        grid=(TN // BM,),
        in_specs=[
            pl.BlockSpec((BM, F0), lambda i: (i, 0)),
            pl.BlockSpec((F0, F1), lambda i: (0, 0)),
            pl.BlockSpec((2 * F1, F2), lambda i: (0, 0)),
        ],
        out_specs=pl.BlockSpec((BM, 2 * F2), lambda i: (i, 0)),
        out_shape=jax.ShapeDtypeStruct((TN, 2 * F2), jnp.float32),
    )(Xf, W1, U)


ZROWS = 200
CROWS = 2000


def _seg_body(aidx, aval, xw, z_out,
              craw_b, didx_b, val_b,
              gidx0, gidx1, gidx2, gidx3,
              didxc0, didxc1, didxc2, didxc3,
              rows0, rows1, rows2, rows3,
              scaled0, scaled1, scaled2, scaled3, zb_v,
              zsh,
              bsem, gsem0, gsem1, gsem2, gsem3,
              ssem0, ssem1, ssem2, ssem3):
    c = lax.axis_index("c")
    s = lax.axis_index("s")
    iota = lax.iota(jnp.int32, L)
    zero = jnp.zeros((L,), jnp.float32)
    gidx = (gidx0, gidx1, gidx2, gidx3)
    didxc = (didxc0, didxc1, didxc2, didxc3)
    rows = (rows0, rows1, rows2, rows3)
    scaled = (scaled0, scaled1, scaled2, scaled3)
    gsem = (gsem0, gsem1, gsem2, gsem3)
    ssem = (ssem0, ssem1, ssem2, ssem3)

    def zfill(i, carry):
        for j in range(F1 // L):
            zb_v[i, pl.ds(j * L, L)] = zero
        return carry

    lax.fori_loop(0, ZROWS, zfill, 0)

    def zero_my_rows():
        @pl.when(s < N // CROWS)
        def _():
            for k in range(CROWS // ZROWS):
                rws = pl.ds(s * CROWS + k * ZROWS, ZROWS)
                pltpu.sync_copy(zb_v, zsh.at[rws])

    zero_my_rows()
    plsc.subcore_barrier()

    for tt in (0, 1):
        t = c * 2 + tt
        tn_vec = jnp.full((L,), t * N, jnp.int32)

        nzs = pl.ds(s * PER_TILE, PER_TILE)
        d1 = pltpu.async_copy(aidx.at[t, 1, nzs], craw_b, bsem)
        d2 = pltpu.async_copy(aidx.at[t, 0, nzs], didx_b, bsem)
        d3 = pltpu.async_copy(aval.at[t, nzs], val_b, bsem)
        d1.wait()
        d2.wait()
        d3.wait()

        def prep_and_gather(j, slot):
            off = j * CH
            for g in range(CH // L):
                d = pl.ds(g * L, L)
                src = pl.ds(off + g * L, L)
                gidx[slot][d] = craw_b[src] + tn_vec
                didxc[slot][d] = didx_b[src]
            pltpu.async_copy(xw.at[gidx[slot]], rows[slot], gsem[slot])

        def wait_gather(slot):
            pltpu.make_async_copy(xw.at[gidx[slot]], rows[slot],
                                  gsem[slot]).wait()

        def scale(j, slot):
            off = j * CH
            rows_s = rows[slot]
            scaled_s = scaled[slot]

            @plsc.parallel_loop(0, CH // L, unroll=CH // L)
            def _(g):
                vv = val_b[pl.ds(off + g * L, L)]
                for r16 in range(L):
                    sv = lax.gather(
                        vv, jnp.full((L, 1), r16, jnp.int32),
                        lax.GatherDimensionNumbers(
                            offset_dims=(), collapsed_slice_dims=(0,),
                            start_index_map=(0,)),
                        (1,), mode=lax.GatherScatterMode.PROMISE_IN_BOUNDS)
                    r = g * L + r16
                    for jb in range(F1 // L):
                        d = pl.ds(jb * L, L)
                        scaled_s[r, d] = rows_s[r, d] * sv

        def scatter_add(slot):
            pltpu.async_copy(scaled[slot], zsh.at[didxc[slot]], ssem[slot],
                             add=True)

        def wait_scatter(slot):
            pltpu.make_async_copy(scaled[slot], zsh.at[didxc[slot]],
                                  ssem[slot]).wait()

        prep_and_gather(0, 0)
        prep_and_gather(1, 1)
        prep_and_gather(2, 2)

        def pipe(m, carry):
            for u in range(4):
                j = 4 * m + u

                @pl.when(j >= 1)
                def _():
                    wait_scatter((u + 3) % 4)

                @pl.when(j + 3 <= NCHUNK - 1)
                def _():
                    prep_and_gather(j + 3, (u + 3) % 4)

                wait_gather(u)
                scale(j, u)
                scatter_add(u)
            return carry

        lax.fori_loop(0, (NCHUNK - 1) // 4, pipe, 0)
        wait_scatter(3)
        wait_gather(0)
        scale(NCHUNK - 1, 0)
        scatter_add(0)
        wait_scatter(0)

        plsc.subcore_barrier()

        @pl.when(s < N // CROWS)
        def _():
            rws = pl.ds(s * CROWS, CROWS)
            pltpu.sync_copy(zsh.at[rws],
                            z_out.at[pl.ds(t * N + s * CROWS, CROWS)])

        if tt == 0:
            zero_my_rows()
            plsc.subcore_barrier()


def _sc_seg(aidx, aval, XW):
    mesh = plsc.VectorSubcoreMesh(core_axis_name="c", subcore_axis_name="s",
                                  num_cores=NC, num_subcores=NS)
    f = pl.kernel(
        _seg_body,
        out_type=jax.ShapeDtypeStruct((TN, F1), jnp.float32),
        mesh=mesh,
        compiler_params=pltpu.CompilerParams(needs_layout_passes=False, use_tc_tiling_on_sc=False),
        scratch_types=[
            pltpu.VMEM((PER_TILE,), jnp.int32),
            pltpu.VMEM((PER_TILE,), jnp.int32),
            pltpu.VMEM((PER_TILE,), jnp.float32),
            pltpu.VMEM((CH,), jnp.int32),
            pltpu.VMEM((CH,), jnp.int32),
            pltpu.VMEM((CH,), jnp.int32),
            pltpu.VMEM((CH,), jnp.int32),
            pltpu.VMEM((CH,), jnp.int32),
            pltpu.VMEM((CH,), jnp.int32),
            pltpu.VMEM((CH,), jnp.int32),
            pltpu.VMEM((CH,), jnp.int32),
            pltpu.VMEM((CH, F1), jnp.float32),
            pltpu.VMEM((CH, F1), jnp.float32),
            pltpu.VMEM((CH, F1), jnp.float32),
            pltpu.VMEM((CH, F1), jnp.float32),
            pltpu.VMEM((CH, F1), jnp.float32),
            pltpu.VMEM((CH, F1), jnp.float32),
            pltpu.VMEM((CH, F1), jnp.float32),
            pltpu.VMEM((CH, F1), jnp.float32),
            pltpu.VMEM((ZROWS, F1), jnp.float32),
            pltpu.VMEM_SHARED((N, F1), jnp.float32),
            pltpu.SemaphoreType.DMA,
            pltpu.SemaphoreType.DMA,
            pltpu.SemaphoreType.DMA,
            pltpu.SemaphoreType.DMA,
            pltpu.SemaphoreType.DMA,
            pltpu.SemaphoreType.DMA,
            pltpu.SemaphoreType.DMA,
            pltpu.SemaphoreType.DMA,
            pltpu.SemaphoreType.DMA,
        ],
    )
    return f(aidx, aval, XW)


def _edge_body(pq, et, es, eg, out,
               tb0, tb1, tb2, sb0, sb1, sb2, gb0, gb1, gb2,
               sidx0, sidx1, sidx2, tidx0, tidx1, tidx2,
               pb0, pb1, pb2, qb0, qb1, qb2, ob0, ob1, ob2,
               isem0, isem1, isem2, psem0, psem1, psem2,
               qsem0, qsem1, qsem2, osem0, osem1, osem2):
    c = lax.axis_index("c")
    s = lax.axis_index("s")
    wid = s * NC + c
    nch = (NCHUNK2 - wid + NW - 1) // NW
    tb = (tb0, tb1, tb2)
    sb = (sb0, sb1, sb2)
    gb = (gb0, gb1, gb2)
    sidx = (sidx0, sidx1, sidx2)
    tidx = (tidx0, tidx1, tidx2)
    pb = (pb0, pb1, pb2)
    qb = (qb0, qb1, qb2)
    ob = (ob0, ob1, ob2)
    isem = (isem0, isem1, isem2)
    psem = (psem0, psem1, psem2)
    qsem = (qsem0, qsem1, qsem2)
    osem = (osem0, osem1, osem2)

    def off_of(k):
        return (wid + k * NW) * CH2

    def idx_load(k, u):
        o = pl.ds(off_of(k), CH2)
        pltpu.async_copy(et.at[o], tb[u], isem[u])
        pltpu.async_copy(es.at[o], sb[u], isem[u])
        pltpu.async_copy(eg.at[o], gb[u], isem[u])

    def idx_wait(u):
        o = pl.ds(0, CH2)
        pltpu.make_async_copy(et.at[o], tb[u], isem[u]).wait()
        pltpu.make_async_copy(es.at[o], sb[u], isem[u]).wait()
        pltpu.make_async_copy(eg.at[o], gb[u], isem[u]).wait()

    def ids_and_gather(u):
        for g in range(CH2 // L):
            d = pl.ds(g * L, L)
            tv = tb[u][d] * N
            sidx[u][d] = tv + sb[u][d]
            tidx[u][d] = tv + gb[u][d]
        pltpu.async_copy(pq.at[sidx[u]], pb[u], psem[u])
        pltpu.async_copy(pq.at[tidx[u]], qb[u], qsem[u])

    def gather_wait(u):
        pltpu.make_async_copy(pq.at[sidx[u]], pb[u], psem[u]).wait()
        pltpu.make_async_copy(pq.at[tidx[u]], qb[u], qsem[u]).wait()

    def out_wait(u):
        pltpu.make_async_copy(ob[u], out.at[pl.ds(0, CH2)], osem[u]).wait()

    idx_load(0, 0)
    idx_load(1, 1)
    idx_wait(0)
    ids_and_gather(0)

    def pipe(m, carry):
        for u in range(3):
            k = 3 * m + u

            @pl.when(k < nch)
            def _():
                gather_wait(u)

                @pl.when(k + 1 < nch)
                def _():
                    idx_wait((u + 1) % 3)
                    ids_and_gather((u + 1) % 3)

                @pl.when(k + 2 < nch)
                def _():
                    idx_load(k + 2, (u + 2) % 3)

                @pl.when(k >= 3)
                def _():
                    out_wait(u)

                for r in range(CH2):
                    for jj in range(F2 // L):
                        dd = pl.ds(jj * L, L)
                        dq = pl.ds(F2 + jj * L, L)
                        ob[u][r, dd] = pb[u][r, dd] + qb[u][r, dq]
                pltpu.async_copy(ob[u], out.at[pl.ds(off_of(k), CH2)],
                                 osem[u])

        return carry

    lax.fori_loop(0, (nch + 2) // 3, pipe, 0)
    out_wait(0)
    out_wait(1)
    out_wait(2)


def _sc_edge(PQ, et, es, eg):
    mesh = plsc.VectorSubcoreMesh(core_axis_name="c", subcore_axis_name="s",
                                  num_cores=NC, num_subcores=NS)
    f = pl.kernel(
        _edge_body,
        out_type=jax.ShapeDtypeStruct((E, F2), jnp.float32),
        mesh=mesh,
        compiler_params=pltpu.CompilerParams(needs_layout_passes=False, use_tc_tiling_on_sc=False),
        scratch_types=(
            [pltpu.VMEM((CH2,), jnp.int32)] * 15
            + [pltpu.VMEM((CH2, 2 * F2), jnp.float32)] * 6
            + [pltpu.VMEM((CH2, F2), jnp.float32)] * 3
            + [pltpu.SemaphoreType.DMA] * 12
        ),
    )
    return f(PQ, et, es, eg)


def kernel(A_idx, A_val, X, edges_t, edges_src, edges_trg, W1, U):
    aidx = A_idx.astype(jnp.int32)
    Xf = X.reshape(TN, F0)
    Y = _tc_y(Xf, W1, U)
    PQ = _sc_seg(aidx, A_val, Y)
    return _sc_edge(PQ,
                    edges_t.astype(jnp.int32),
                    edges_src.astype(jnp.int32),
                    edges_trg.astype(jnp.int32))


# speedup vs baseline: 1.0382x; 1.0002x over previous
import jax
import jax.numpy as jnp
from jax import lax
from jax.experimental import pallas as pl
from jax.experimental.pallas import tpu as pltpu
from jax.experimental.pallas import tpu_sc as plsc

T, N, F0, F1, F2 = 4, 10000, 128, 64, 32
NNZ, E = 160000, 200000
NC, NS, L = 2, 16, 16
NW = NC * NS
TN = T * N

PER_TILE = NNZ // NS
CH = 80
NCHUNK = PER_TILE // CH
RPT = N // NS

CH2 = 80
NCHUNK2 = E // CH2


def _mm_y_body(x_ref, w_ref, u_ref, o_ref):
    w = w_ref[...]
    g = jnp.concatenate(
        [jnp.dot(w, u_ref[:F1, :], preferred_element_type=jnp.float32),
         jnp.dot(w, u_ref[F1:, :], preferred_element_type=jnp.float32)],
        axis=1)
    o_ref[...] = jnp.dot(x_ref[...], g, preferred_element_type=jnp.float32)


def _tc_y(Xf, W1, U):
    BM = 2000
    return pl.pallas_call(
        _mm_y_body,
        grid=(TN // BM,),
        in_specs=[
            pl.BlockSpec((BM, F0), lambda i: (i, 0)),
            pl.BlockSpec((F0, F1), lambda i: (0, 0)),
            pl.BlockSpec((2 * F1, F2), lambda i: (0, 0)),
        ],
        out_specs=pl.BlockSpec((BM, 2 * F2), lambda i: (i, 0)),
        out_shape=jax.ShapeDtypeStruct((TN, 2 * F2), jnp.float32),
    )(Xf, W1, U)


ZROWS = 200
CROWS = 2000


def _seg_body(aidx, aval, xw, z_out,
              craw_b, didx_b, val_b,
              gidx0, gidx1, gidx2, gidx3,
              didxc0, didxc1, didxc2, didxc3,
              rows0, rows1, rows2, rows3,
              scaled0, scaled1, scaled2, scaled3, zb_v,
              zsh,
              bsem, gsem0, gsem1, gsem2, gsem3,
              ssem0, ssem1, ssem2, ssem3):
    c = lax.axis_index("c")
    s = lax.axis_index("s")
    iota = lax.iota(jnp.int32, L)
    zero = jnp.zeros((L,), jnp.float32)
    gidx = (gidx0, gidx1, gidx2, gidx3)
    didxc = (didxc0, didxc1, didxc2, didxc3)
    rows = (rows0, rows1, rows2, rows3)
    scaled = (scaled0, scaled1, scaled2, scaled3)
    gsem = (gsem0, gsem1, gsem2, gsem3)
    ssem = (ssem0, ssem1, ssem2, ssem3)

    def zfill(i, carry):
        for j in range(F1 // L):
            zb_v[i, pl.ds(j * L, L)] = zero
        return carry

    lax.fori_loop(0, ZROWS, zfill, 0)

    def zero_my_rows():
        @pl.when(s < N // CROWS)
        def _():
            for k in range(CROWS // ZROWS):
                rws = pl.ds(s * CROWS + k * ZROWS, ZROWS)
                pltpu.sync_copy(zb_v, zsh.at[rws])

    zero_my_rows()
    plsc.subcore_barrier()

    for tt in (0, 1):
        t = c * 2 + tt
        tn_vec = jnp.full((L,), t * N, jnp.int32)

        nzs = pl.ds(s * PER_TILE, PER_TILE)
        d1 = pltpu.async_copy(aidx.at[t, 1, nzs], craw_b, bsem)
        d2 = pltpu.async_copy(aidx.at[t, 0, nzs], didx_b, bsem)
        d3 = pltpu.async_copy(aval.at[t, nzs], val_b, bsem)
        d1.wait()
        d2.wait()
        d3.wait()

        def prep_and_gather(j, slot):
            off = j * CH
            for g in range(CH // L):
                d = pl.ds(g * L, L)
                src = pl.ds(off + g * L, L)
                gidx[slot][d] = craw_b[src] + tn_vec
                didxc[slot][d] = didx_b[src]
            pltpu.async_copy(xw.at[gidx[slot]], rows[slot], gsem[slot])

        def wait_gather(slot):
            pltpu.make_async_copy(xw.at[gidx[slot]], rows[slot],
                                  gsem[slot]).wait()

        def scale(j, slot):
            off = j * CH
            rows_s = rows[slot]
            scaled_s = scaled[slot]

            @plsc.parallel_loop(0, CH // L, unroll=CH // L)
            def _(g):
                vv = val_b[pl.ds(off + g * L, L)]
                for r16 in range(L):
                    sv = lax.gather(
                        vv, jnp.full((L, 1), r16, jnp.int32),
                        lax.GatherDimensionNumbers(
                            offset_dims=(), collapsed_slice_dims=(0,),
                            start_index_map=(0,)),
                        (1,), mode=lax.GatherScatterMode.PROMISE_IN_BOUNDS)
                    r = g * L + r16
                    for jb in range(F1 // L):
                        d = pl.ds(jb * L, L)
                        scaled_s[r, d] = rows_s[r, d] * sv

        def scatter_add(slot):
            pltpu.async_copy(scaled[slot], zsh.at[didxc[slot]], ssem[slot],
                             add=True)

        def wait_scatter(slot):
            pltpu.make_async_copy(scaled[slot], zsh.at[didxc[slot]],
                                  ssem[slot]).wait()

        prep_and_gather(0, 0)
        prep_and_gather(1, 1)
        prep_and_gather(2, 2)

        def pipe(m, carry):
            for u in range(4):
                j = 4 * m + u

                @pl.when(j >= 1)
                def _():
                    wait_scatter((u + 3) % 4)

                @pl.when(j + 3 <= NCHUNK - 1)
                def _():
                    prep_and_gather(j + 3, (u + 3) % 4)

                wait_gather(u)
                scale(j, u)
                scatter_add(u)
            return carry

        lax.fori_loop(0, (NCHUNK - 1) // 4, pipe, 0)
        wait_scatter(3)
        wait_gather(0)
        scale(NCHUNK - 1, 0)
        scatter_add(0)
        wait_scatter(0)

        plsc.subcore_barrier()

        @pl.when(s < N // CROWS)
        def _():
            rws = pl.ds(s * CROWS, CROWS)
            pltpu.sync_copy(zsh.at[rws],
                            z_out.at[pl.ds(t * N + s * CROWS, CROWS)])

        if tt == 0:
            zero_my_rows()
            plsc.subcore_barrier()


def _sc_seg(aidx, aval, XW):
    mesh = plsc.VectorSubcoreMesh(core_axis_name="c", subcore_axis_name="s",
                                  num_cores=NC, num_subcores=NS)
    f = pl.kernel(
        _seg_body,
        out_type=jax.ShapeDtypeStruct((TN, F1), jnp.float32),
        mesh=mesh,
        compiler_params=pltpu.CompilerParams(needs_layout_passes=False, use_tc_tiling_on_sc=False),
        scratch_types=[
            pltpu.VMEM((PER_TILE,), jnp.int32),
            pltpu.VMEM((PER_TILE,), jnp.int32),
            pltpu.VMEM((PER_TILE,), jnp.float32),
            pltpu.VMEM((CH,), jnp.int32),
            pltpu.VMEM((CH,), jnp.int32),
            pltpu.VMEM((CH,), jnp.int32),
            pltpu.VMEM((CH,), jnp.int32),
            pltpu.VMEM((CH,), jnp.int32),
            pltpu.VMEM((CH,), jnp.int32),
            pltpu.VMEM((CH,), jnp.int32),
            pltpu.VMEM((CH,), jnp.int32),
            pltpu.VMEM((CH, F1), jnp.float32),
            pltpu.VMEM((CH, F1), jnp.float32),
            pltpu.VMEM((CH, F1), jnp.float32),
            pltpu.VMEM((CH, F1), jnp.float32),
            pltpu.VMEM((CH, F1), jnp.float32),
            pltpu.VMEM((CH, F1), jnp.float32),
            pltpu.VMEM((CH, F1), jnp.float32),
            pltpu.VMEM((CH, F1), jnp.float32),
            pltpu.VMEM((ZROWS, F1), jnp.float32),
            pltpu.VMEM_SHARED((N, F1), jnp.float32),
            pltpu.SemaphoreType.DMA,
            pltpu.SemaphoreType.DMA,
            pltpu.SemaphoreType.DMA,
            pltpu.SemaphoreType.DMA,
            pltpu.SemaphoreType.DMA,
            pltpu.SemaphoreType.DMA,
            pltpu.SemaphoreType.DMA,
            pltpu.SemaphoreType.DMA,
            pltpu.SemaphoreType.DMA,
        ],
    )
    return f(aidx, aval, XW)


def _edge_body(pq, et, es, eg, out,
               tb0, tb1, tb2, sb0, sb1, sb2, gb0, gb1, gb2,
               sidx0, sidx1, sidx2, tidx0, tidx1, tidx2,
               pb0, pb1, pb2, qb0, qb1, qb2, ob0, ob1, ob2,
               isem0, isem1, isem2, psem0, psem1, psem2,
               qsem0, qsem1, qsem2, osem0, osem1, osem2):
    c = lax.axis_index("c")
    s = lax.axis_index("s")
    wid = s * NC + c
    nch = (NCHUNK2 - wid + NW - 1) // NW
    tb = (tb0, tb1, tb2)
    sb = (sb0, sb1, sb2)
    gb = (gb0, gb1, gb2)
    sidx = (sidx0, sidx1, sidx2)
    tidx = (tidx0, tidx1, tidx2)
    pb = (pb0, pb1, pb2)
    qb = (qb0, qb1, qb2)
    ob = (ob0, ob1, ob2)
    isem = (isem0, isem1, isem2)
    psem = (psem0, psem1, psem2)
    qsem = (qsem0, qsem1, qsem2)
    osem = (osem0, osem1, osem2)

    def off_of(k):
        return (wid + k * NW) * CH2

    def idx_load(k, u):
        o = pl.ds(off_of(k), CH2)
        pltpu.async_copy(et.at[o], tb[u], isem[u])
        pltpu.async_copy(es.at[o], sb[u], isem[u])
        pltpu.async_copy(eg.at[o], gb[u], isem[u])

    def idx_wait(u):
        o = pl.ds(0, CH2)
        pltpu.make_async_copy(et.at[o], tb[u], isem[u]).wait()
        pltpu.make_async_copy(es.at[o], sb[u], isem[u]).wait()
        pltpu.make_async_copy(eg.at[o], gb[u], isem[u]).wait()

    def ids_and_gather(u):
        for g in range(CH2 // L):
            d = pl.ds(g * L, L)
            tv = tb[u][d] * N
            sidx[u][d] = tv + sb[u][d]
            tidx[u][d] = tv + gb[u][d]
        pltpu.async_copy(pq.at[sidx[u]], pb[u], psem[u])
        pltpu.async_copy(pq.at[tidx[u]], qb[u], qsem[u])

    def gather_wait(u):
        pltpu.make_async_copy(pq.at[sidx[u]], pb[u], psem[u]).wait()
        pltpu.make_async_copy(pq.at[tidx[u]], qb[u], qsem[u]).wait()

    def out_wait(u):
        pltpu.make_async_copy(ob[u], out.at[pl.ds(0, CH2)], osem[u]).wait()

    idx_load(0, 0)
    idx_load(1, 1)
    idx_wait(0)
    ids_and_gather(0)

    def pipe(m, carry):
        for u in range(3):
            k = 3 * m + u

            @pl.when(k < nch)
            def _():
                gather_wait(u)

                @pl.when(k + 1 < nch)
                def _():
                    idx_wait((u + 1) % 3)
                    ids_and_gather((u + 1) % 3)

                @pl.when(k + 2 < nch)
                def _():
                    idx_load(k + 2, (u + 2) % 3)

                @pl.when(k >= 3)
                def _():
                    out_wait(u)

                for r in range(CH2):
                    for jj in range(F2 // L):
                        dd = pl.ds(jj * L, L)
                        dq = pl.ds(F2 + jj * L, L)
                        ob[u][r, dd] = pb[u][r, dd] + qb[u][r, dq]
                pltpu.async_copy(ob[u], out.at[pl.ds(off_of(k), CH2)],
                                 osem[u])

        return carry

    lax.fori_loop(0, (nch + 2) // 3, pipe, 0)
    out_wait(0)
    out_wait(1)
    out_wait(2)


def _sc_edge(PQ, et, es, eg):
    mesh = plsc.VectorSubcoreMesh(core_axis_name="c", subcore_axis_name="s",
                                  num_cores=NC, num_subcores=NS)
    f = pl.kernel(
        _edge_body,
        out_type=jax.ShapeDtypeStruct((E, F2), jnp.float32),
        mesh=mesh,
        compiler_params=pltpu.CompilerParams(needs_layout_passes=False, use_tc_tiling_on_sc=False),
        scratch_types=(
            [pltpu.VMEM((CH2,), jnp.int32)] * 15
            + [pltpu.VMEM((CH2, 2 * F2), jnp.float32)] * 6
            + [pltpu.VMEM((CH2, F2), jnp.float32)] * 3
            + [pltpu.SemaphoreType.DMA] * 12
        ),
    )
    return f(PQ, et, es, eg)


def kernel(A_idx, A_val, X, edges_t, edges_src, edges_trg, W1, U):
    aidx = A_idx.astype(jnp.int32)
    Xf = X.reshape(TN, F0)
    Y = _tc_y(Xf, W1, U)
    PQ = _sc_seg(aidx, A_val, Y)
    return _sc_edge(PQ,
                    edges_t.astype(jnp.int32),
                    edges_src.astype(jnp.int32),
                    edges_trg.astype(jnp.int32))
